# baseline (speedup 1.0000x reference)
import functools

import jax
import jax.numpy as jnp
from jax import lax
from jax.experimental import pallas as pl
from jax.experimental.pallas import tpu as pltpu
from jax.experimental.pallas import tpu_sc as plsc

N_NODES = 20000
N_EDGES = 40000
MAX_NB = 4
VOCAB = 40
MAX_POS = 20
N_BONDS = 4
HIDDEN = 256
DEPTH = 5
ATOM_FDIM = VOCAB + MAX_POS
INPUT_SIZE = ATOM_FDIM + N_BONDS
HALF = HIDDEN // 2

E2 = 40960
BU = 1024
BR = 1000

_F32 = jnp.float32

_NW = 32
_NBUF = 4
_WIN = 64
_SKEW = 0.95


def _sc_gather_rows(table, idx_2d):
    ts, win = idx_2d.shape
    assert win == _WIN
    D = table.shape[1]
    per16 = ts // 16
    quant = max(_NBUF, 8)
    n0 = int(round(per16 * _SKEW / quant)) * quant
    n0 = max(quant, min(n0, per16 - quant))
    n1 = per16 - n0
    mesh = plsc.VectorSubcoreMesh(core_axis_name="core",
                                  subcore_axis_name="subcore")

    @functools.partial(
        pl.kernel,
        out_type=jax.ShapeDtypeStruct((ts * _WIN, D), table.dtype),
        mesh=mesh,
        scratch_types=(
            [pltpu.VMEM((max(n0, n1), _WIN), jnp.int32),
             pltpu.VMEM((_NBUF, _WIN, D), table.dtype)]
            + [pltpu.SemaphoreType.DMA] * (2 * _NBUF)
        ),
    )
    def k(x_hbm, i_hbm, o_hbm, idx_v, bufs, *sems):
        gsem = sems[:_NBUF]
        wsem = sems[_NBUF:]
        cid = lax.axis_index("core")
        sid = lax.axis_index("subcore")

        def run(nsteps, sbase):
            pltpu.sync_copy(i_hbm.at[pl.ds(sbase, nsteps)],
                            idx_v.at[pl.ds(0, nsteps)])
            obase = sbase * _WIN

            def start_gather(g, b):
                pltpu.async_copy(x_hbm.at[idx_v.at[g]], bufs.at[b], gsem[b])

            for b in range(_NBUF):
                start_gather(b, b)

            @pl.loop(0, nsteps // _NBUF)
            def _(s):
                g0 = s * _NBUF
                for b in range(_NBUF):
                    pltpu.make_async_copy(x_hbm.at[idx_v.at[0]], bufs.at[b],
                                          gsem[b]).wait()
                    dst = o_hbm.at[pl.ds(obase + (g0 + b) * _WIN, _WIN)]
                    pltpu.async_copy(bufs.at[b], dst, wsem[b])
                    pltpu.make_async_copy(bufs.at[b], dst, wsem[b]).wait()

                    @pl.when(g0 + b + _NBUF < nsteps)
                    def _():
                        start_gather(g0 + b + _NBUF, b)

        @pl.when(cid == 0)
        def _():
            run(n0, sid * n0)

        @pl.when(cid == 1)
        def _():
            run(n1, 16 * n0 + sid * n1)

    return k(table, idx_2d)


def _sigmoid(x):
    return 1.0 / (1.0 + jnp.exp(-x))


def _pack2(lo, hi):
    lob = lax.bitcast_convert_type(lo, jnp.int32) + 0x8000
    hib = lax.bitcast_convert_type(hi, jnp.int32) + 0x8000
    lo16 = lax.shift_right_logical(lob, 16)
    hi16 = jnp.bitwise_and(hib, jnp.int32(-65536))
    return jnp.bitwise_or(lo16, hi16)


def _pack_cols(x):
    return _pack2(x[:, 0:HALF], x[:, HALF:])


def _unpack_cols(w):
    lo = lax.bitcast_convert_type(jnp.left_shift(w, 16), _F32)
    hi = lax.bitcast_convert_type(jnp.bitwise_and(w, jnp.int32(-65536)), _F32)
    return jnp.concatenate([lo, hi], axis=1)


def _edge_pre_kernel(fsrc, fmess, wz1, wr, wh1, bz, bh, out):
    a = fsrc[:, 0:1]
    p = fsrc[:, 1:2]
    bnd = fmess[:, 2:3]
    be = a.shape[0]
    io = lax.broadcasted_iota(jnp.int32, (be, INPUT_SIZE), 1)
    x = ((io == a) | (io == p + VOCAB) | (io == bnd + ATOM_FDIM)).astype(_F32)
    pz = jnp.dot(x, wz1[...], preferred_element_type=_F32) + bz[...]
    nr = -jnp.dot(x, wr[...], preferred_element_type=_F32)
    ph = jnp.dot(x, wh1[...], preferred_element_type=_F32) + bh[...]
    out[:, 0:HALF] = _pack_cols(pz)
    out[:, HALF:2 * HALF] = _pack_cols(nr)
    out[:, 2 * HALF:] = _pack_cols(ph)


def _node_pre_kernel(fnode, wo1, bo, out):
    a = fnode[:, 0:1]
    p = fnode[:, 1:2]
    bn = a.shape[0]
    io = lax.broadcasted_iota(jnp.int32, (bn, ATOM_FDIM), 1)
    x = ((io == a) | (io == p + VOCAB)).astype(_F32)
    out[...] = jnp.dot(x, wo1[...], preferred_element_type=_F32) + bo[...]


def _row_mask(block):
    gid = pl.program_id(0)
    return lax.broadcasted_iota(jnp.int32, (block, 1), 0) + gid * block > 0


def _pack_state(h, ur, bur, out):
    hu = jnp.dot(h, ur, preferred_element_type=_F32) + bur
    out[:, 0:HALF] = _pack_cols(h)
    out[:, HALF:] = _pack_cols(hu)


def _first_iter_kernel(pzh, ur, bur, out, *, block):
    z = _sigmoid(_unpack_cols(pzh[:, 0:HALF]))
    ph = jnp.tanh(_unpack_cols(pzh[:, 2 * HALF:]))
    h = jnp.where(_row_mask(block), z * ph, 0.0)
    _pack_state(h, ur[...], bur[...], out)


def _update_kernel(g0, g1, g2, g3, pzh, wz2, wh2, ur, bur, out,
                   *, block, with_hu):
    nr1 = _unpack_cols(pzh[:, HALF:2 * HALF])
    sum_h = None
    sgh = None
    for g in (g0, g1, g2, g3):
        hk = _unpack_cols(g[:, 0:HALF])
        uk = _unpack_cols(g[:, HALF:])
        sk = hk / (1.0 + jnp.exp(nr1 - uk))
        sum_h = hk if sum_h is None else sum_h + hk
        sgh = sk if sgh is None else sgh + sk
    z = _sigmoid(_unpack_cols(pzh[:, 0:HALF])
                 + jnp.dot(sum_h, wz2[...], preferred_element_type=_F32))
    ph = jnp.tanh(_unpack_cols(pzh[:, 2 * HALF:])
                  + jnp.dot(sgh, wh2[...], preferred_element_type=_F32))
    h = (1.0 - z) * sum_h + z * ph
    h = jnp.where(_row_mask(block), h, 0.0)
    if with_hu:
        _pack_state(h, ur[...], bur[...], out)
    else:
        out[...] = _pack_cols(h)


def _readout_kernel(g0, g1, g2, g3, pre_o, wo2, out, *, block):
    nei = None
    for g in (g0, g1, g2, g3):
        hk = _unpack_cols(g[...])
        nei = hk if nei is None else nei + hk
    y = pre_o[...] + jnp.dot(nei, wo2[...], preferred_element_type=_F32)
    y = jnp.maximum(y, 0.0)
    out[...] = jnp.where(_row_mask(block), y, 0.0)


def _full(spec_shape):
    return pl.BlockSpec(spec_shape, lambda i: tuple(0 for _ in spec_shape))


def _seg_idx(idx, rows, seg_rows):
    padded = jnp.pad(idx, ((0, seg_rows - idx.shape[0]), (0, 0)))
    flat = padded.T.reshape(-1)
    return flat.reshape(-1, _WIN)


def kernel(fnode, fmess, agraph, bgraph, scope, W_z, b_z, W_r, U_r, b_ur,
           W_h, b_h, W_o, b_o):
    del scope
    perm = jnp.concatenate([jnp.arange(0, HIDDEN, 2, dtype=jnp.int32),
                            jnp.arange(1, HIDDEN, 2, dtype=jnp.int32)])
    Wz1, Wz2 = W_z[:INPUT_SIZE][:, perm], W_z[INPUT_SIZE:][perm][:, perm]
    Wh1, Wh2 = W_h[:INPUT_SIZE][:, perm], W_h[INPUT_SIZE:][perm][:, perm]
    Wo1, Wo2 = W_o[:ATOM_FDIM], W_o[ATOM_FDIM:][perm]
    Urp = U_r[perm][:, perm]
    Wrp = W_r[:, perm]
    bz = b_z[perm].reshape(1, HIDDEN)
    bh = b_h[perm].reshape(1, HIDDEN)
    bo = b_o.reshape(1, HIDDEN)
    bur = b_ur[perm].reshape(1, HIDDEN)

    fmess_pad = jnp.pad(fmess, ((0, E2 - N_EDGES), (0, 0)))
    src_idx = fmess_pad[:, 0].reshape(-1, _WIN)
    bg_idx = _seg_idx(bgraph, N_EDGES, E2)
    ag_flat = agraph.T.reshape(-1)
    ag_pad = 16 * _NBUF * _WIN * 2
    ag_total = ((ag_flat.shape[0] + ag_pad - 1) // ag_pad) * ag_pad
    ag_idx = jnp.pad(ag_flat, (0, ag_total - ag_flat.shape[0]))
    ag_idx = ag_idx.reshape(-1, _WIN)
    fnode_pad = jnp.pad(fnode, ((0, 0), (0, 126)))

    fsrc = _sc_gather_rows(fnode_pad, src_idx)

    pzh = pl.pallas_call(
        _edge_pre_kernel,
        grid=(E2 // BU,),
        in_specs=[
            pl.BlockSpec((BU, 128), lambda i: (i, 0)),
            pl.BlockSpec((BU, 3), lambda i: (i, 0)),
            _full((INPUT_SIZE, HIDDEN)),
            _full((INPUT_SIZE, HIDDEN)),
            _full((INPUT_SIZE, HIDDEN)),
            _full((1, HIDDEN)),
            _full((1, HIDDEN)),
        ],
        out_specs=pl.BlockSpec((BU, 3 * HALF), lambda i: (i, 0)),
        out_shape=jax.ShapeDtypeStruct((E2, 3 * HALF), jnp.int32),
    )(fsrc, fmess_pad, Wz1, Wrp, Wh1, bz, bh)

    pre_o = pl.pallas_call(
        _node_pre_kernel,
        grid=(N_NODES // BR,),
        in_specs=[
            pl.BlockSpec((BR, 2), lambda i: (i, 0)),
            _full((ATOM_FDIM, HIDDEN)),
            _full((1, HIDDEN)),
        ],
        out_specs=pl.BlockSpec((BR, HIDDEN), lambda i: (i, 0)),
        out_shape=jax.ShapeDtypeStruct((N_NODES, HIDDEN), _F32),
    )(fnode, Wo1, bo)

    c = pl.pallas_call(
        functools.partial(_first_iter_kernel, block=BU),
        grid=(E2 // BU,),
        in_specs=[
            pl.BlockSpec((BU, 3 * HALF), lambda i: (i, 0)),
            _full((HIDDEN, HIDDEN)),
            _full((1, HIDDEN)),
        ],
        out_specs=pl.BlockSpec((BU, HIDDEN), lambda i: (i, 0)),
        out_shape=jax.ShapeDtypeStruct((E2, HIDDEN), jnp.int32),
    )(pzh, Urp, bur)

    nseg = E2 // BU
    for it in range(DEPTH - 1):
        with_hu = it < DEPTH - 2
        g = _sc_gather_rows(c, bg_idx)
        out_w = HIDDEN if with_hu else HALF
        c = pl.pallas_call(
            functools.partial(_update_kernel, block=BU, with_hu=with_hu),
            grid=(E2 // BU,),
            in_specs=[
                pl.BlockSpec((BU, HIDDEN),
                             functools.partial(lambda k, i: (i + k * nseg, 0), k))
                for k in range(MAX_NB)
            ] + [
                pl.BlockSpec((BU, 3 * HALF), lambda i: (i, 0)),
                _full((HIDDEN, HIDDEN)),
                _full((HIDDEN, HIDDEN)),
                _full((HIDDEN, HIDDEN)),
                _full((1, HIDDEN)),
            ],
            out_specs=pl.BlockSpec((BU, out_w), lambda i: (i, 0)),
            out_shape=jax.ShapeDtypeStruct((E2, out_w), jnp.int32),
        )(g, g, g, g, pzh, Wz2, Wh2, Urp, bur)

    gn = _sc_gather_rows(c, ag_idx)
    nsegn = N_NODES // BR
    out = pl.pallas_call(
        functools.partial(_readout_kernel, block=BR),
        grid=(N_NODES // BR,),
        in_specs=[
            pl.BlockSpec((BR, HALF),
                         functools.partial(lambda k, i: (i + k * nsegn, 0), k))
            for k in range(MAX_NB)
        ] + [
            pl.BlockSpec((BR, HIDDEN), lambda i: (i, 0)),
            _full((HIDDEN, HIDDEN)),
        ],
        out_specs=pl.BlockSpec((BR, HIDDEN), lambda i: (i, 0)),
        out_shape=jax.ShapeDtypeStruct((N_NODES, HIDDEN), _F32),
    )(gn, gn, gn, gn, pre_o, Wo2)
    return out


# speedup vs baseline: 1.3374x; 1.3374x over previous
import functools

import jax
import jax.numpy as jnp
from jax import lax
from jax.experimental import pallas as pl
from jax.experimental.pallas import tpu as pltpu
from jax.experimental.pallas import tpu_sc as plsc

N_NODES = 20000
N_EDGES = 40000
MAX_NB = 4
VOCAB = 40
MAX_POS = 20
N_BONDS = 4
HIDDEN = 256
DEPTH = 5
ATOM_FDIM = VOCAB + MAX_POS
INPUT_SIZE = ATOM_FDIM + N_BONDS
HALF = HIDDEN // 2

E2 = 40960
BU = 1024
BR = 1000

_F32 = jnp.float32

_NW = 32
_NBUF = 4
_WIN = 64
_SKEW = 0.75


def _sc_gather_rows(table, idx_2d):
    ts, win = idx_2d.shape
    assert win == _WIN
    D = table.shape[1]
    per16 = ts // 16
    quant = max(_NBUF, 8)
    n0 = int(round(per16 * _SKEW / quant)) * quant
    n0 = max(quant, min(n0, per16 - quant))
    n1 = per16 - n0
    mesh = plsc.VectorSubcoreMesh(core_axis_name="core",
                                  subcore_axis_name="subcore")

    @functools.partial(
        pl.kernel,
        out_type=jax.ShapeDtypeStruct((ts * _WIN, D), table.dtype),
        mesh=mesh,
        scratch_types=(
            [pltpu.VMEM((max(n0, n1), _WIN), jnp.int32),
             pltpu.VMEM((_NBUF, _WIN, D), table.dtype)]
            + [pltpu.SemaphoreType.DMA] * (2 * _NBUF)
        ),
    )
    def k(x_hbm, i_hbm, o_hbm, idx_v, bufs, *sems):
        gsem = sems[:_NBUF]
        wsem = sems[_NBUF:]
        cid = lax.axis_index("core")
        sid = lax.axis_index("subcore")

        def run(nsteps, sbase):
            pltpu.sync_copy(i_hbm.at[pl.ds(sbase, nsteps)],
                            idx_v.at[pl.ds(0, nsteps)])
            obase = sbase * _WIN

            def start_gather(g, b):
                pltpu.async_copy(x_hbm.at[idx_v.at[g]], bufs.at[b], gsem[b])

            for b in range(_NBUF):
                start_gather(b, b)

            @pl.loop(0, nsteps // _NBUF)
            def _(s):
                g0 = s * _NBUF
                for b in range(_NBUF):
                    pltpu.make_async_copy(x_hbm.at[idx_v.at[0]], bufs.at[b],
                                          gsem[b]).wait()
                    dst = o_hbm.at[pl.ds(obase + (g0 + b) * _WIN, _WIN)]
                    pltpu.async_copy(bufs.at[b], dst, wsem[b])
                    pltpu.make_async_copy(bufs.at[b], dst, wsem[b]).wait()

                    @pl.when(g0 + b + _NBUF < nsteps)
                    def _():
                        start_gather(g0 + b + _NBUF, b)

        @pl.when(cid == 0)
        def _():
            run(n0, sid * n0)

        @pl.when(cid == 1)
        def _():
            run(n1, 16 * n0 + sid * n1)

    return k(table, idx_2d)


def _sigmoid(x):
    return 1.0 / (1.0 + jnp.exp(-x))


def _pack2(lo, hi):
    lob = lax.bitcast_convert_type(lo, jnp.int32) + 0x8000
    hib = lax.bitcast_convert_type(hi, jnp.int32) + 0x8000
    lo16 = lax.shift_right_logical(lob, 16)
    hi16 = jnp.bitwise_and(hib, jnp.int32(-65536))
    return jnp.bitwise_or(lo16, hi16)


def _pack_cols(x):
    return _pack2(x[:, 0:HALF], x[:, HALF:])


def _unpack_cols(w):
    lo = lax.bitcast_convert_type(jnp.left_shift(w, 16), _F32)
    hi = lax.bitcast_convert_type(jnp.bitwise_and(w, jnp.int32(-65536)), _F32)
    return jnp.concatenate([lo, hi], axis=1)


def _edge_pre_kernel(fsrc, fmess, wz1, wr, wh1, bz, bh, out):
    a = fsrc[:, 0:1]
    p = fsrc[:, 1:2]
    bnd = fmess[:, 2:3]
    be = a.shape[0]
    io = lax.broadcasted_iota(jnp.int32, (be, INPUT_SIZE), 1)
    x = ((io == a) | (io == p + VOCAB) | (io == bnd + ATOM_FDIM)).astype(_F32)
    pz = jnp.dot(x, wz1[...], preferred_element_type=_F32) + bz[...]
    nr = -jnp.dot(x, wr[...], preferred_element_type=_F32)
    ph = jnp.dot(x, wh1[...], preferred_element_type=_F32) + bh[...]
    out[:, 0:HALF] = _pack_cols(pz)
    out[:, HALF:2 * HALF] = _pack_cols(nr)
    out[:, 2 * HALF:] = _pack_cols(ph)


def _node_pre_kernel(fnode, wo1, bo, out):
    a = fnode[:, 0:1]
    p = fnode[:, 1:2]
    bn = a.shape[0]
    io = lax.broadcasted_iota(jnp.int32, (bn, ATOM_FDIM), 1)
    x = ((io == a) | (io == p + VOCAB)).astype(_F32)
    out[...] = jnp.dot(x, wo1[...], preferred_element_type=_F32) + bo[...]


def _row_mask(block):
    gid = pl.program_id(0)
    return lax.broadcasted_iota(jnp.int32, (block, 1), 0) + gid * block > 0


def _first_iter_kernel(pzh, out, *, block):
    z = _sigmoid(_unpack_cols(pzh[:, 0:HALF]))
    ph = jnp.tanh(_unpack_cols(pzh[:, 2 * HALF:]))
    h = jnp.where(_row_mask(block), z * ph, 0.0)
    out[...] = _pack_cols(h)


_BF16 = jnp.bfloat16


def _update_kernel(g0, g1, g2, g3, pzh, wz2, wh2, ur, bur, out, *, block):
    nr1 = _unpack_cols(pzh[:, HALF:2 * HALF])
    sum_h = None
    sgh = None
    for g in (g0, g1, g2, g3):
        hk = _unpack_cols(g[...])
        uk = jnp.dot(hk.astype(_BF16), ur[...],
                     preferred_element_type=_F32) + bur[...]
        sk = hk / (1.0 + jnp.exp(nr1 - uk))
        sum_h = hk if sum_h is None else sum_h + hk
        sgh = sk if sgh is None else sgh + sk
    z = _sigmoid(_unpack_cols(pzh[:, 0:HALF])
                 + jnp.dot(sum_h.astype(_BF16), wz2[...],
                           preferred_element_type=_F32))
    ph = jnp.tanh(_unpack_cols(pzh[:, 2 * HALF:])
                  + jnp.dot(sgh.astype(_BF16), wh2[...],
                            preferred_element_type=_F32))
    h = (1.0 - z) * sum_h + z * ph
    h = jnp.where(_row_mask(block), h, 0.0)
    out[...] = _pack_cols(h)


def _readout_kernel(g0, g1, g2, g3, pre_o, wo2, out, *, block):
    nei = None
    for g in (g0, g1, g2, g3):
        hk = _unpack_cols(g[...])
        nei = hk if nei is None else nei + hk
    y = pre_o[...] + jnp.dot(nei.astype(_BF16), wo2[...],
                             preferred_element_type=_F32)
    y = jnp.maximum(y, 0.0)
    out[...] = jnp.where(_row_mask(block), y, 0.0)


def _full(spec_shape):
    return pl.BlockSpec(spec_shape, lambda i: tuple(0 for _ in spec_shape))


def _seg_idx(idx, rows, seg_rows):
    padded = jnp.pad(idx, ((0, seg_rows - idx.shape[0]), (0, 0)))
    flat = padded.T.reshape(-1)
    return flat.reshape(-1, _WIN)


def kernel(fnode, fmess, agraph, bgraph, scope, W_z, b_z, W_r, U_r, b_ur,
           W_h, b_h, W_o, b_o):
    del scope
    perm = jnp.concatenate([jnp.arange(0, HIDDEN, 2, dtype=jnp.int32),
                            jnp.arange(1, HIDDEN, 2, dtype=jnp.int32)])
    Wz1, Wz2 = W_z[:INPUT_SIZE][:, perm], W_z[INPUT_SIZE:][perm][:, perm]
    Wh1, Wh2 = W_h[:INPUT_SIZE][:, perm], W_h[INPUT_SIZE:][perm][:, perm]
    Wo1, Wo2 = W_o[:ATOM_FDIM], W_o[ATOM_FDIM:][perm]
    Urp = U_r[perm][:, perm]
    Wrp = W_r[:, perm]
    bz = b_z[perm].reshape(1, HIDDEN)
    bh = b_h[perm].reshape(1, HIDDEN)
    bo = b_o.reshape(1, HIDDEN)
    bur = b_ur[perm].reshape(1, HIDDEN)

    fmess_pad = jnp.pad(fmess, ((0, E2 - N_EDGES), (0, 0)))
    src_idx = fmess_pad[:, 0].reshape(-1, _WIN)
    bg_idx = _seg_idx(bgraph, N_EDGES, E2)
    ag_flat = agraph.T.reshape(-1)
    ag_pad = 16 * _NBUF * _WIN * 2
    ag_total = ((ag_flat.shape[0] + ag_pad - 1) // ag_pad) * ag_pad
    ag_idx = jnp.pad(ag_flat, (0, ag_total - ag_flat.shape[0]))
    ag_idx = ag_idx.reshape(-1, _WIN)
    fnode_pad = jnp.pad(fnode, ((0, 0), (0, 126)))

    fsrc = _sc_gather_rows(fnode_pad, src_idx)

    pzh = pl.pallas_call(
        _edge_pre_kernel,
        grid=(E2 // BU,),
        in_specs=[
            pl.BlockSpec((BU, 128), lambda i: (i, 0)),
            pl.BlockSpec((BU, 3), lambda i: (i, 0)),
            _full((INPUT_SIZE, HIDDEN)),
            _full((INPUT_SIZE, HIDDEN)),
            _full((INPUT_SIZE, HIDDEN)),
            _full((1, HIDDEN)),
            _full((1, HIDDEN)),
        ],
        out_specs=pl.BlockSpec((BU, 3 * HALF), lambda i: (i, 0)),
        out_shape=jax.ShapeDtypeStruct((E2, 3 * HALF), jnp.int32),
    )(fsrc, fmess_pad, Wz1, Wrp, Wh1, bz, bh)

    pre_o = pl.pallas_call(
        _node_pre_kernel,
        grid=(N_NODES // BR,),
        in_specs=[
            pl.BlockSpec((BR, 2), lambda i: (i, 0)),
            _full((ATOM_FDIM, HIDDEN)),
            _full((1, HIDDEN)),
        ],
        out_specs=pl.BlockSpec((BR, HIDDEN), lambda i: (i, 0)),
        out_shape=jax.ShapeDtypeStruct((N_NODES, HIDDEN), _F32),
    )(fnode, Wo1, bo)

    c = pl.pallas_call(
        functools.partial(_first_iter_kernel, block=BU),
        grid=(E2 // BU,),
        in_specs=[pl.BlockSpec((BU, 3 * HALF), lambda i: (i, 0))],
        out_specs=pl.BlockSpec((BU, HALF), lambda i: (i, 0)),
        out_shape=jax.ShapeDtypeStruct((E2, HALF), jnp.int32),
    )(pzh)

    Wz2b = Wz2.astype(_BF16)
    Wh2b = Wh2.astype(_BF16)
    Urpb = Urp.astype(_BF16)
    nseg = E2 // BU
    for _ in range(DEPTH - 1):
        g = _sc_gather_rows(c, bg_idx)
        c = pl.pallas_call(
            functools.partial(_update_kernel, block=BU),
            grid=(E2 // BU,),
            in_specs=[
                pl.BlockSpec((BU, HALF),
                             functools.partial(lambda k, i: (i + k * nseg, 0), k))
                for k in range(MAX_NB)
            ] + [
                pl.BlockSpec((BU, 3 * HALF), lambda i: (i, 0)),
                _full((HIDDEN, HIDDEN)),
                _full((HIDDEN, HIDDEN)),
                _full((HIDDEN, HIDDEN)),
                _full((1, HIDDEN)),
            ],
            out_specs=pl.BlockSpec((BU, HALF), lambda i: (i, 0)),
            out_shape=jax.ShapeDtypeStruct((E2, HALF), jnp.int32),
        )(g, g, g, g, pzh, Wz2b, Wh2b, Urpb, bur)

    gn = _sc_gather_rows(c, ag_idx)
    nsegn = N_NODES // BR
    out = pl.pallas_call(
        functools.partial(_readout_kernel, block=BR),
        grid=(N_NODES // BR,),
        in_specs=[
            pl.BlockSpec((BR, HALF),
                         functools.partial(lambda k, i: (i + k * nsegn, 0), k))
            for k in range(MAX_NB)
        ] + [
            pl.BlockSpec((BR, HIDDEN), lambda i: (i, 0)),
            _full((HIDDEN, HIDDEN)),
        ],
        out_specs=pl.BlockSpec((BR, HIDDEN), lambda i: (i, 0)),
        out_shape=jax.ShapeDtypeStruct((N_NODES, HIDDEN), _F32),
    )(gn, gn, gn, gn, pre_o, Wo2.astype(_BF16))
    return out


# speedup vs baseline: 1.3609x; 1.0176x over previous
import functools

import jax
import jax.numpy as jnp
from jax import lax
from jax.experimental import pallas as pl
from jax.experimental.pallas import tpu as pltpu
from jax.experimental.pallas import tpu_sc as plsc

N_NODES = 20000
N_EDGES = 40000
MAX_NB = 4
VOCAB = 40
MAX_POS = 20
N_BONDS = 4
HIDDEN = 256
DEPTH = 5
ATOM_FDIM = VOCAB + MAX_POS
INPUT_SIZE = ATOM_FDIM + N_BONDS
HALF = HIDDEN // 2

E2 = 40960
BU = 1024
BR = 1000

_F32 = jnp.float32

_NW = 32
_NBUF = 8
_WIN = 64
_SKEW = 0.70


def _sc_gather_rows(table, idx_2d, skew=_SKEW):
    ts, win = idx_2d.shape
    assert win == _WIN
    D = table.shape[1]
    per16 = ts // 16
    quant = max(_NBUF, 8)
    if skew >= 1.0:
        n0, n1 = per16, 0
    else:
        n0 = int(round(per16 * skew / quant)) * quant
        n0 = max(quant, min(n0, per16 - quant))
        n1 = per16 - n0
    assert n0 % quant == 0 and n1 % quant == 0
    mesh = plsc.VectorSubcoreMesh(core_axis_name="core",
                                  subcore_axis_name="subcore")

    @functools.partial(
        pl.kernel,
        out_type=jax.ShapeDtypeStruct((ts * _WIN, D), table.dtype),
        mesh=mesh,
        scratch_types=(
            [pltpu.VMEM((max(n0, n1), _WIN), jnp.int32),
             pltpu.VMEM((_NBUF, _WIN, D), table.dtype)]
            + [pltpu.SemaphoreType.DMA] * (2 * _NBUF)
        ),
    )
    def k(x_hbm, i_hbm, o_hbm, idx_v, bufs, *sems):
        gsem = sems[:_NBUF]
        wsem = sems[_NBUF:]
        cid = lax.axis_index("core")
        sid = lax.axis_index("subcore")

        def run(nsteps, sbase):
            pltpu.sync_copy(i_hbm.at[pl.ds(sbase, nsteps)],
                            idx_v.at[pl.ds(0, nsteps)])
            obase = sbase * _WIN

            def start_gather(g, b):
                pltpu.async_copy(x_hbm.at[idx_v.at[g]], bufs.at[b], gsem[b])

            for b in range(_NBUF):
                start_gather(b, b)

            @pl.loop(0, nsteps // _NBUF)
            def _(s):
                g0 = s * _NBUF
                for b in range(_NBUF):
                    pltpu.make_async_copy(x_hbm.at[idx_v.at[0]], bufs.at[b],
                                          gsem[b]).wait()
                    dst = o_hbm.at[pl.ds(obase + (g0 + b) * _WIN, _WIN)]
                    pltpu.async_copy(bufs.at[b], dst, wsem[b])
                    pltpu.make_async_copy(bufs.at[b], dst, wsem[b]).wait()

                    @pl.when(g0 + b + _NBUF < nsteps)
                    def _():
                        start_gather(g0 + b + _NBUF, b)

        @pl.when(cid == 0)
        def _():
            run(n0, sid * n0)

        if n1 > 0:
            @pl.when(cid == 1)
            def _():
                run(n1, 16 * n0 + sid * n1)

    return k(table, idx_2d)


def _sigmoid(x):
    return 1.0 / (1.0 + jnp.exp(-x))


def _pack2(lo, hi):
    lob = lax.bitcast_convert_type(lo, jnp.int32) + 0x8000
    hib = lax.bitcast_convert_type(hi, jnp.int32) + 0x8000
    lo16 = lax.shift_right_logical(lob, 16)
    hi16 = jnp.bitwise_and(hib, jnp.int32(-65536))
    return jnp.bitwise_or(lo16, hi16)


def _pack_cols(x):
    return _pack2(x[:, 0:HALF], x[:, HALF:])


def _unpack_cols(w):
    lo = lax.bitcast_convert_type(jnp.left_shift(w, 16), _F32)
    hi = lax.bitcast_convert_type(jnp.bitwise_and(w, jnp.int32(-65536)), _F32)
    return jnp.concatenate([lo, hi], axis=1)


def _edge_pre_kernel(fsrc, fmess, wz1, wr, wh1, bz, bh, out):
    a = fsrc[:, 0:1]
    p = fsrc[:, 1:2]
    bnd = fmess[:, 2:3]
    be = a.shape[0]
    io = lax.broadcasted_iota(jnp.int32, (be, INPUT_SIZE), 1)
    x = ((io == a) | (io == p + VOCAB) | (io == bnd + ATOM_FDIM)).astype(_F32)
    pz = jnp.dot(x, wz1[...], preferred_element_type=_F32) + bz[...]
    nr = -jnp.dot(x, wr[...], preferred_element_type=_F32)
    ph = jnp.dot(x, wh1[...], preferred_element_type=_F32) + bh[...]
    out[:, 0:HALF] = _pack_cols(pz)
    out[:, HALF:2 * HALF] = _pack_cols(nr)
    out[:, 2 * HALF:] = _pack_cols(ph)


def _node_pre_kernel(fnode, wo1, bo, out):
    a = fnode[:, 0:1]
    p = fnode[:, 1:2]
    bn = a.shape[0]
    io = lax.broadcasted_iota(jnp.int32, (bn, ATOM_FDIM), 1)
    x = ((io == a) | (io == p + VOCAB)).astype(_F32)
    out[...] = jnp.dot(x, wo1[...], preferred_element_type=_F32) + bo[...]


def _row_mask(block):
    gid = pl.program_id(0)
    return lax.broadcasted_iota(jnp.int32, (block, 1), 0) + gid * block > 0


def _first_iter_kernel(pzh, out, *, block):
    z = _sigmoid(_unpack_cols(pzh[:, 0:HALF]))
    ph = jnp.tanh(_unpack_cols(pzh[:, 2 * HALF:]))
    h = jnp.where(_row_mask(block), z * ph, 0.0)
    out[...] = _pack_cols(h)


_BF16 = jnp.bfloat16


def _update_kernel(g0, g1, g2, g3, pzh, wz2, wh2, ur, bur, out, *, block):
    nr1 = _unpack_cols(pzh[:, HALF:2 * HALF])
    sum_h = None
    sgh = None
    for g in (g0, g1, g2, g3):
        hk = _unpack_cols(g[...])
        uk = jnp.dot(hk.astype(_BF16), ur[...],
                     preferred_element_type=_F32) + bur[...]
        sk = hk / (1.0 + jnp.exp(nr1 - uk))
        sum_h = hk if sum_h is None else sum_h + hk
        sgh = sk if sgh is None else sgh + sk
    z = _sigmoid(_unpack_cols(pzh[:, 0:HALF])
                 + jnp.dot(sum_h.astype(_BF16), wz2[...],
                           preferred_element_type=_F32))
    ph = jnp.tanh(_unpack_cols(pzh[:, 2 * HALF:])
                  + jnp.dot(sgh.astype(_BF16), wh2[...],
                            preferred_element_type=_F32))
    h = (1.0 - z) * sum_h + z * ph
    h = jnp.where(_row_mask(block), h, 0.0)
    out[...] = _pack_cols(h)


def _readout_kernel(g0, g1, g2, g3, pre_o, wo2, out, *, block):
    nei = None
    for g in (g0, g1, g2, g3):
        hk = _unpack_cols(g[...])
        nei = hk if nei is None else nei + hk
    y = pre_o[...] + jnp.dot(nei.astype(_BF16), wo2[...],
                             preferred_element_type=_F32)
    y = jnp.maximum(y, 0.0)
    out[...] = jnp.where(_row_mask(block), y, 0.0)


def _full(spec_shape):
    return pl.BlockSpec(spec_shape, lambda i: tuple(0 for _ in spec_shape))


def _seg_idx(idx, rows, seg_rows):
    padded = jnp.pad(idx, ((0, seg_rows - idx.shape[0]), (0, 0)))
    flat = padded.T.reshape(-1)
    return flat.reshape(-1, _WIN)


def kernel(fnode, fmess, agraph, bgraph, scope, W_z, b_z, W_r, U_r, b_ur,
           W_h, b_h, W_o, b_o):
    del scope
    perm = jnp.concatenate([jnp.arange(0, HIDDEN, 2, dtype=jnp.int32),
                            jnp.arange(1, HIDDEN, 2, dtype=jnp.int32)])
    Wz1, Wz2 = W_z[:INPUT_SIZE][:, perm], W_z[INPUT_SIZE:][perm][:, perm]
    Wh1, Wh2 = W_h[:INPUT_SIZE][:, perm], W_h[INPUT_SIZE:][perm][:, perm]
    Wo1, Wo2 = W_o[:ATOM_FDIM], W_o[ATOM_FDIM:][perm]
    Urp = U_r[perm][:, perm]
    Wrp = W_r[:, perm]
    bz = b_z[perm].reshape(1, HIDDEN)
    bh = b_h[perm].reshape(1, HIDDEN)
    bo = b_o.reshape(1, HIDDEN)
    bur = b_ur[perm].reshape(1, HIDDEN)

    fmess_pad = jnp.pad(fmess, ((0, E2 - N_EDGES), (0, 0)))
    src_idx = fmess_pad[:, 0].reshape(-1, _WIN)
    bg_idx = _seg_idx(bgraph, N_EDGES, E2)
    ag_flat = agraph.T.reshape(-1)
    ag_pad = 16 * _NBUF * _WIN * 2
    ag_total = ((ag_flat.shape[0] + ag_pad - 1) // ag_pad) * ag_pad
    ag_idx = jnp.pad(ag_flat, (0, ag_total - ag_flat.shape[0]))
    ag_idx = ag_idx.reshape(-1, _WIN)
    fnode_pad = jnp.pad(fnode, ((0, 0), (0, 126)))

    fsrc = _sc_gather_rows(fnode_pad, src_idx, skew=1.0)

    pzh = pl.pallas_call(
        _edge_pre_kernel,
        grid=(E2 // BU,),
        in_specs=[
            pl.BlockSpec((BU, 128), lambda i: (i, 0)),
            pl.BlockSpec((BU, 3), lambda i: (i, 0)),
            _full((INPUT_SIZE, HIDDEN)),
            _full((INPUT_SIZE, HIDDEN)),
            _full((INPUT_SIZE, HIDDEN)),
            _full((1, HIDDEN)),
            _full((1, HIDDEN)),
        ],
        out_specs=pl.BlockSpec((BU, 3 * HALF), lambda i: (i, 0)),
        out_shape=jax.ShapeDtypeStruct((E2, 3 * HALF), jnp.int32),
    )(fsrc, fmess_pad, Wz1, Wrp, Wh1, bz, bh)

    pre_o = pl.pallas_call(
        _node_pre_kernel,
        grid=(N_NODES // BR,),
        in_specs=[
            pl.BlockSpec((BR, 2), lambda i: (i, 0)),
            _full((ATOM_FDIM, HIDDEN)),
            _full((1, HIDDEN)),
        ],
        out_specs=pl.BlockSpec((BR, HIDDEN), lambda i: (i, 0)),
        out_shape=jax.ShapeDtypeStruct((N_NODES, HIDDEN), _F32),
    )(fnode, Wo1, bo)

    c = pl.pallas_call(
        functools.partial(_first_iter_kernel, block=BU),
        grid=(E2 // BU,),
        in_specs=[pl.BlockSpec((BU, 3 * HALF), lambda i: (i, 0))],
        out_specs=pl.BlockSpec((BU, HALF), lambda i: (i, 0)),
        out_shape=jax.ShapeDtypeStruct((E2, HALF), jnp.int32),
    )(pzh)

    Wz2b = Wz2.astype(_BF16)
    Wh2b = Wh2.astype(_BF16)
    Urpb = Urp.astype(_BF16)
    nseg = E2 // BU
    for _ in range(DEPTH - 1):
        g = _sc_gather_rows(c, bg_idx)
        c = pl.pallas_call(
            functools.partial(_update_kernel, block=BU),
            grid=(E2 // BU,),
            in_specs=[
                pl.BlockSpec((BU, HALF),
                             functools.partial(lambda k, i: (i + k * nseg, 0), k))
                for k in range(MAX_NB)
            ] + [
                pl.BlockSpec((BU, 3 * HALF), lambda i: (i, 0)),
                _full((HIDDEN, HIDDEN)),
                _full((HIDDEN, HIDDEN)),
                _full((HIDDEN, HIDDEN)),
                _full((1, HIDDEN)),
            ],
            out_specs=pl.BlockSpec((BU, HALF), lambda i: (i, 0)),
            out_shape=jax.ShapeDtypeStruct((E2, HALF), jnp.int32),
        )(g, g, g, g, pzh, Wz2b, Wh2b, Urpb, bur)

    gn = _sc_gather_rows(c, ag_idx, skew=1.0)
    nsegn = N_NODES // BR
    out = pl.pallas_call(
        functools.partial(_readout_kernel, block=BR),
        grid=(N_NODES // BR,),
        in_specs=[
            pl.BlockSpec((BR, HALF),
                         functools.partial(lambda k, i: (i + k * nsegn, 0), k))
            for k in range(MAX_NB)
        ] + [
            pl.BlockSpec((BR, HIDDEN), lambda i: (i, 0)),
            _full((HIDDEN, HIDDEN)),
        ],
        out_specs=pl.BlockSpec((BR, HIDDEN), lambda i: (i, 0)),
        out_shape=jax.ShapeDtypeStruct((N_NODES, HIDDEN), _F32),
    )(gn, gn, gn, gn, pre_o, Wo2.astype(_BF16))
    return out


# speedup vs baseline: 1.4820x; 1.0889x over previous
import functools

import jax
import jax.numpy as jnp
from jax import lax
from jax.experimental import pallas as pl
from jax.experimental.pallas import tpu as pltpu
from jax.experimental.pallas import tpu_sc as plsc

N_NODES = 20000
N_EDGES = 40000
MAX_NB = 4
VOCAB = 40
MAX_POS = 20
N_BONDS = 4
HIDDEN = 256
DEPTH = 5
ATOM_FDIM = VOCAB + MAX_POS
INPUT_SIZE = ATOM_FDIM + N_BONDS
HALF = HIDDEN // 2

E2 = 40960
BU = 1024
BR = 1000

_F32 = jnp.float32

_NW = 32
_NBUF = 8
_WIN = 64
_SKEW = 0.70


def _sc_gather_rows(table, idx_2d, skew=_SKEW):
    ts, win = idx_2d.shape
    assert win == _WIN
    D = table.shape[1]
    per16 = ts // 16
    quant = max(_NBUF, 8)
    if skew >= 1.0:
        n0, n1 = per16, 0
    else:
        n0 = int(round(per16 * skew / quant)) * quant
        n0 = max(quant, min(n0, per16 - quant))
        n1 = per16 - n0
    assert n0 % quant == 0 and n1 % quant == 0
    mesh = plsc.VectorSubcoreMesh(core_axis_name="core",
                                  subcore_axis_name="subcore")

    @functools.partial(
        pl.kernel,
        out_type=jax.ShapeDtypeStruct((ts * _WIN, D), table.dtype),
        mesh=mesh,
        scratch_types=(
            [pltpu.VMEM((max(n0, n1), _WIN), jnp.int32),
             pltpu.VMEM((_NBUF, _WIN, D), table.dtype)]
            + [pltpu.SemaphoreType.DMA] * (2 * _NBUF)
        ),
    )
    def k(x_hbm, i_hbm, o_hbm, idx_v, bufs, *sems):
        gsem = sems[:_NBUF]
        wsem = sems[_NBUF:]
        cid = lax.axis_index("core")
        sid = lax.axis_index("subcore")

        def run(nsteps, sbase):
            pltpu.sync_copy(i_hbm.at[pl.ds(sbase, nsteps)],
                            idx_v.at[pl.ds(0, nsteps)])
            obase = sbase * _WIN

            def start_gather(g, b):
                pltpu.async_copy(x_hbm.at[idx_v.at[g]], bufs.at[b], gsem[b])

            for b in range(_NBUF):
                start_gather(b, b)

            @pl.loop(0, nsteps // _NBUF)
            def _(s):
                g0 = s * _NBUF
                for b in range(_NBUF):
                    pltpu.make_async_copy(x_hbm.at[idx_v.at[0]], bufs.at[b],
                                          gsem[b]).wait()
                    dst = o_hbm.at[pl.ds(obase + (g0 + b) * _WIN, _WIN)]
                    pltpu.async_copy(bufs.at[b], dst, wsem[b])
                    pltpu.make_async_copy(bufs.at[b], dst, wsem[b]).wait()

                    @pl.when(g0 + b + _NBUF < nsteps)
                    def _():
                        start_gather(g0 + b + _NBUF, b)

        @pl.when(cid == 0)
        def _():
            run(n0, sid * n0)

        if n1 > 0:
            @pl.when(cid == 1)
            def _():
                run(n1, 16 * n0 + sid * n1)

    return k(table, idx_2d)


def _sigmoid(x):
    return 1.0 / (1.0 + jnp.exp(-x))


def _pack2(lo, hi):
    lob = lax.bitcast_convert_type(lo, jnp.int32) + 0x8000
    hib = lax.bitcast_convert_type(hi, jnp.int32) + 0x8000
    lo16 = lax.shift_right_logical(lob, 16)
    hi16 = jnp.bitwise_and(hib, jnp.int32(-65536))
    return jnp.bitwise_or(lo16, hi16)


def _pack_cols(x):
    return _pack2(x[:, 0:HALF], x[:, HALF:])


def _unpack_cols(w):
    lo = lax.bitcast_convert_type(jnp.left_shift(w, 16), _F32)
    hi = lax.bitcast_convert_type(jnp.bitwise_and(w, jnp.int32(-65536)), _F32)
    return jnp.concatenate([lo, hi], axis=1)


def _edge_pre_kernel(fsrc, fmess, wz1, wr, wh1, bz, bh, out):
    a = fsrc[:, 0:1]
    p = fsrc[:, 1:2]
    bnd = fmess[:, 2:3]
    be = a.shape[0]
    io = lax.broadcasted_iota(jnp.int32, (be, INPUT_SIZE), 1)
    x = ((io == a) | (io == p + VOCAB) | (io == bnd + ATOM_FDIM)).astype(_F32)
    pz = jnp.dot(x, wz1[...], preferred_element_type=_F32) + bz[...]
    nr = -jnp.dot(x, wr[...], preferred_element_type=_F32)
    ph = jnp.dot(x, wh1[...], preferred_element_type=_F32) + bh[...]
    out[:, 0:HALF] = _pack_cols(pz)
    out[:, HALF:2 * HALF] = _pack_cols(nr)
    out[:, 2 * HALF:] = _pack_cols(ph)


def _node_pre_kernel(fnode, wo1, bo, out):
    a = fnode[:, 0:1]
    p = fnode[:, 1:2]
    bn = a.shape[0]
    io = lax.broadcasted_iota(jnp.int32, (bn, ATOM_FDIM), 1)
    x = ((io == a) | (io == p + VOCAB)).astype(_F32)
    out[...] = jnp.dot(x, wo1[...], preferred_element_type=_F32) + bo[...]


def _row_mask(block, blk_off=0):
    gid = pl.program_id(0) + blk_off
    return lax.broadcasted_iota(jnp.int32, (block, 1), 0) + gid * block > 0


def _first_iter_kernel(pzh, out, *, block):
    z = _sigmoid(_unpack_cols(pzh[:, 0:HALF]))
    ph = jnp.tanh(_unpack_cols(pzh[:, 2 * HALF:]))
    h = jnp.where(_row_mask(block), z * ph, 0.0)
    out[...] = _pack_cols(h)


_BF16 = jnp.bfloat16


def _update_kernel(g0, g1, g2, g3, pzh, wz2, wh2, ur, bur, *refs,
                   block, blk_off):
    out = refs[-1]
    nr1 = _unpack_cols(pzh[:, HALF:2 * HALF])
    sum_h = None
    sgh = None
    for g in (g0, g1, g2, g3):
        hk = _unpack_cols(g[...])
        uk = jnp.dot(hk.astype(_BF16), ur[...],
                     preferred_element_type=_F32) + bur[...]
        sk = hk / (1.0 + jnp.exp(nr1 - uk))
        sum_h = hk if sum_h is None else sum_h + hk
        sgh = sk if sgh is None else sgh + sk
    z = _sigmoid(_unpack_cols(pzh[:, 0:HALF])
                 + jnp.dot(sum_h.astype(_BF16), wz2[...],
                           preferred_element_type=_F32))
    ph = jnp.tanh(_unpack_cols(pzh[:, 2 * HALF:])
                  + jnp.dot(sgh.astype(_BF16), wh2[...],
                            preferred_element_type=_F32))
    h = (1.0 - z) * sum_h + z * ph
    h = jnp.where(_row_mask(block, blk_off), h, 0.0)
    out[...] = _pack_cols(h)


def _readout_kernel(g0, g1, g2, g3, pre_o, wo2, out, *, block):
    nei = None
    for g in (g0, g1, g2, g3):
        hk = _unpack_cols(g[...])
        nei = hk if nei is None else nei + hk
    y = pre_o[...] + jnp.dot(nei.astype(_BF16), wo2[...],
                             preferred_element_type=_F32)
    y = jnp.maximum(y, 0.0)
    out[...] = jnp.where(_row_mask(block), y, 0.0)


def _full(spec_shape):
    return pl.BlockSpec(spec_shape, lambda i: tuple(0 for _ in spec_shape))


def _seg_idx(idx, rows, seg_rows):
    padded = jnp.pad(idx, ((0, seg_rows - idx.shape[0]), (0, 0)))
    flat = padded.T.reshape(-1)
    return flat.reshape(-1, _WIN)


def kernel(fnode, fmess, agraph, bgraph, scope, W_z, b_z, W_r, U_r, b_ur,
           W_h, b_h, W_o, b_o):
    del scope
    perm = jnp.concatenate([jnp.arange(0, HIDDEN, 2, dtype=jnp.int32),
                            jnp.arange(1, HIDDEN, 2, dtype=jnp.int32)])
    Wz1, Wz2 = W_z[:INPUT_SIZE][:, perm], W_z[INPUT_SIZE:][perm][:, perm]
    Wh1, Wh2 = W_h[:INPUT_SIZE][:, perm], W_h[INPUT_SIZE:][perm][:, perm]
    Wo1, Wo2 = W_o[:ATOM_FDIM], W_o[ATOM_FDIM:][perm]
    Urp = U_r[perm][:, perm]
    Wrp = W_r[:, perm]
    bz = b_z[perm].reshape(1, HIDDEN)
    bh = b_h[perm].reshape(1, HIDDEN)
    bo = b_o.reshape(1, HIDDEN)
    bur = b_ur[perm].reshape(1, HIDDEN)

    fmess_pad = jnp.pad(fmess, ((0, E2 - N_EDGES), (0, 0)))
    src_idx = fmess_pad[:, 0].reshape(-1, _WIN)
    EH = E2 // 2
    bgraph_pad = jnp.pad(bgraph, ((0, E2 - N_EDGES), (0, 0)))
    bgA_idx = bgraph_pad[:EH].T.reshape(-1, _WIN)
    bgB_idx = bgraph_pad[EH:].T.reshape(-1, _WIN)
    ag_flat = agraph.T.reshape(-1)
    ag_pad = 16 * _NBUF * _WIN * 2
    ag_total = ((ag_flat.shape[0] + ag_pad - 1) // ag_pad) * ag_pad
    ag_idx = jnp.pad(ag_flat, (0, ag_total - ag_flat.shape[0]))
    ag_idx = ag_idx.reshape(-1, _WIN)
    fnode_pad = jnp.pad(fnode, ((0, 0), (0, 126)))

    fsrc = _sc_gather_rows(fnode_pad, src_idx, skew=1.0)

    pzh = pl.pallas_call(
        _edge_pre_kernel,
        grid=(E2 // BU,),
        in_specs=[
            pl.BlockSpec((BU, 128), lambda i: (i, 0)),
            pl.BlockSpec((BU, 3), lambda i: (i, 0)),
            _full((INPUT_SIZE, HIDDEN)),
            _full((INPUT_SIZE, HIDDEN)),
            _full((INPUT_SIZE, HIDDEN)),
            _full((1, HIDDEN)),
            _full((1, HIDDEN)),
        ],
        out_specs=pl.BlockSpec((BU, 3 * HALF), lambda i: (i, 0)),
        out_shape=jax.ShapeDtypeStruct((E2, 3 * HALF), jnp.int32),
    )(fsrc, fmess_pad, Wz1, Wrp, Wh1, bz, bh)

    pre_o = pl.pallas_call(
        _node_pre_kernel,
        grid=(N_NODES // BR,),
        in_specs=[
            pl.BlockSpec((BR, 2), lambda i: (i, 0)),
            _full((ATOM_FDIM, HIDDEN)),
            _full((1, HIDDEN)),
        ],
        out_specs=pl.BlockSpec((BR, HIDDEN), lambda i: (i, 0)),
        out_shape=jax.ShapeDtypeStruct((N_NODES, HIDDEN), _F32),
    )(fnode, Wo1, bo)

    c = pl.pallas_call(
        functools.partial(_first_iter_kernel, block=BU),
        grid=(E2 // BU,),
        in_specs=[pl.BlockSpec((BU, 3 * HALF), lambda i: (i, 0))],
        out_specs=pl.BlockSpec((BU, HALF), lambda i: (i, 0)),
        out_shape=jax.ShapeDtypeStruct((E2, HALF), jnp.int32),
    )(pzh)

    Wz2b = Wz2.astype(_BF16)
    Wh2b = Wh2.astype(_BF16)
    Urpb = Urp.astype(_BF16)
    nsegh = EH // BU

    def upd_half(g, half, prev):
        blk_off = half * nsegh
        gspecs = [
            pl.BlockSpec((BU, HALF),
                         functools.partial(lambda k, i: (i + k * nsegh, 0), k))
            for k in range(MAX_NB)
        ]
        dspec = pl.BlockSpec((BU, HALF), lambda i: (i + blk_off, 0))
        other = [
            pl.BlockSpec((BU, 3 * HALF), lambda i: (i + blk_off, 0)),
            _full((HIDDEN, HIDDEN)),
            _full((HIDDEN, HIDDEN)),
            _full((HIDDEN, HIDDEN)),
            _full((1, HIDDEN)),
        ]
        args = (g, g, g, g, pzh, Wz2b, Wh2b, Urpb, bur)
        aliases = {}
        if prev is not None:
            other = other + [dspec]
            args = args + (prev,)
            aliases = {9: 0}
        return pl.pallas_call(
            functools.partial(_update_kernel, block=BU, blk_off=blk_off),
            grid=(EH // BU,),
            in_specs=gspecs + other,
            out_specs=dspec,
            out_shape=jax.ShapeDtypeStruct((E2, HALF), jnp.int32),
            input_output_aliases=aliases,
        )(*args)

    for _ in range(DEPTH - 1):
        ga = _sc_gather_rows(c, bgA_idx, skew=1.0)
        gb = _sc_gather_rows(c, bgB_idx, skew=1.0)
        ca = upd_half(ga, 0, None)
        c = upd_half(gb, 1, ca)

    gn = _sc_gather_rows(c, ag_idx, skew=1.0)
    nsegn = N_NODES // BR
    out = pl.pallas_call(
        functools.partial(_readout_kernel, block=BR),
        grid=(N_NODES // BR,),
        in_specs=[
            pl.BlockSpec((BR, HALF),
                         functools.partial(lambda k, i: (i + k * nsegn, 0), k))
            for k in range(MAX_NB)
        ] + [
            pl.BlockSpec((BR, HIDDEN), lambda i: (i, 0)),
            _full((HIDDEN, HIDDEN)),
        ],
        out_specs=pl.BlockSpec((BR, HIDDEN), lambda i: (i, 0)),
        out_shape=jax.ShapeDtypeStruct((N_NODES, HIDDEN), _F32),
    )(gn, gn, gn, gn, pre_o, Wo2.astype(_BF16))
    return out


# speedup vs baseline: 1.4823x; 1.0002x over previous
import functools

import jax
import jax.numpy as jnp
from jax import lax
from jax.experimental import pallas as pl
from jax.experimental.pallas import tpu as pltpu
from jax.experimental.pallas import tpu_sc as plsc

N_NODES = 20000
N_EDGES = 40000
MAX_NB = 4
VOCAB = 40
MAX_POS = 20
N_BONDS = 4
HIDDEN = 256
DEPTH = 5
ATOM_FDIM = VOCAB + MAX_POS
INPUT_SIZE = ATOM_FDIM + N_BONDS
HALF = HIDDEN // 2

E2 = 40960
BU = 1024
BR = 1000

_F32 = jnp.float32

_NW = 32
_NBUF = 8
_WIN = 64
_SKEW = 0.70


def _sc_gather_rows(table, idx_2d, skew=_SKEW):
    ts, win = idx_2d.shape
    assert win == _WIN
    D = table.shape[1]
    per16 = ts // 16
    quant = max(_NBUF, 8)
    if skew >= 1.0:
        n0, n1 = per16, 0
    else:
        n0 = int(round(per16 * skew / quant)) * quant
        n0 = max(quant, min(n0, per16 - quant))
        n1 = per16 - n0
    assert n0 % quant == 0 and n1 % quant == 0
    mesh = plsc.VectorSubcoreMesh(core_axis_name="core",
                                  subcore_axis_name="subcore")

    @functools.partial(
        pl.kernel,
        out_type=jax.ShapeDtypeStruct((ts * _WIN, D), table.dtype),
        mesh=mesh,
        scratch_types=(
            [pltpu.VMEM((max(n0, n1), _WIN), jnp.int32),
             pltpu.VMEM((_NBUF, _WIN, D), table.dtype)]
            + [pltpu.SemaphoreType.DMA] * (2 * _NBUF)
        ),
    )
    def k(x_hbm, i_hbm, o_hbm, idx_v, bufs, *sems):
        gsem = sems[:_NBUF]
        wsem = sems[_NBUF:]
        cid = lax.axis_index("core")
        sid = lax.axis_index("subcore")

        def run(nsteps, sbase):
            pltpu.sync_copy(i_hbm.at[pl.ds(sbase, nsteps)],
                            idx_v.at[pl.ds(0, nsteps)])
            obase = sbase * _WIN

            def start_gather(g, b):
                pltpu.async_copy(x_hbm.at[idx_v.at[g]], bufs.at[b], gsem[b])

            for b in range(_NBUF):
                start_gather(b, b)

            @pl.loop(0, nsteps // _NBUF)
            def _(s):
                g0 = s * _NBUF
                for b in range(_NBUF):
                    pltpu.make_async_copy(x_hbm.at[idx_v.at[0]], bufs.at[b],
                                          gsem[b]).wait()
                    dst = o_hbm.at[pl.ds(obase + (g0 + b) * _WIN, _WIN)]
                    pltpu.async_copy(bufs.at[b], dst, wsem[b])
                    pltpu.make_async_copy(bufs.at[b], dst, wsem[b]).wait()

                    @pl.when(g0 + b + _NBUF < nsteps)
                    def _():
                        start_gather(g0 + b + _NBUF, b)

        @pl.when(cid == 0)
        def _():
            run(n0, sid * n0)

        if n1 > 0:
            @pl.when(cid == 1)
            def _():
                run(n1, 16 * n0 + sid * n1)

    return k(table, idx_2d)


def _sigmoid(x):
    return 1.0 / (1.0 + jnp.exp(-x))


def _pack2(lo, hi):
    lob = lax.bitcast_convert_type(lo, jnp.int32) + 0x8000
    hib = lax.bitcast_convert_type(hi, jnp.int32) + 0x8000
    lo16 = lax.shift_right_logical(lob, 16)
    hi16 = jnp.bitwise_and(hib, jnp.int32(-65536))
    return jnp.bitwise_or(lo16, hi16)


def _pack_cols(x):
    return _pack2(x[:, 0:HALF], x[:, HALF:])


def _unpack_cols(w):
    lo = lax.bitcast_convert_type(jnp.left_shift(w, 16), _F32)
    hi = lax.bitcast_convert_type(jnp.bitwise_and(w, jnp.int32(-65536)), _F32)
    return jnp.concatenate([lo, hi], axis=1)


def _edge_pre_kernel(fsrc, fmess, wz1, wr, wh1, bz, bh, out):
    a = fsrc[:, 0:1]
    p = fsrc[:, 1:2]
    bnd = fmess[:, 2:3]
    be = a.shape[0]
    io = lax.broadcasted_iota(jnp.int32, (be, INPUT_SIZE), 1)
    x = ((io == a) | (io == p + VOCAB) | (io == bnd + ATOM_FDIM)).astype(_F32)
    pz = jnp.dot(x, wz1[...], preferred_element_type=_F32) + bz[...]
    nr = -jnp.dot(x, wr[...], preferred_element_type=_F32)
    ph = jnp.dot(x, wh1[...], preferred_element_type=_F32) + bh[...]
    out[:, 0:HALF] = _pack_cols(pz)
    out[:, HALF:2 * HALF] = _pack_cols(nr)
    out[:, 2 * HALF:] = _pack_cols(ph)


def _node_pre_kernel(fnode, wo1, bo, out):
    a = fnode[:, 0:1]
    p = fnode[:, 1:2]
    bn = a.shape[0]
    io = lax.broadcasted_iota(jnp.int32, (bn, ATOM_FDIM), 1)
    x = ((io == a) | (io == p + VOCAB)).astype(_F32)
    out[...] = jnp.dot(x, wo1[...], preferred_element_type=_F32) + bo[...]


def _row_mask(block, blk_off=0):
    gid = pl.program_id(0) + blk_off
    return lax.broadcasted_iota(jnp.int32, (block, 1), 0) + gid * block > 0


def _first_iter_kernel(pzh, out, *, block):
    z = _sigmoid(_unpack_cols(pzh[:, 0:HALF]))
    ph = jnp.tanh(_unpack_cols(pzh[:, 2 * HALF:]))
    h = jnp.where(_row_mask(block), z * ph, 0.0)
    out[...] = _pack_cols(h)


_BF16 = jnp.bfloat16


def _update_kernel(g0, g1, g2, g3, pzh, wz2, wh2, ur, bur, *refs,
                   block, blk_off):
    out = refs[-1]
    nr1 = _unpack_cols(pzh[:, HALF:2 * HALF])
    sum_h = None
    sgh = None
    for g in (g0, g1, g2, g3):
        hk = _unpack_cols(g[...])
        uk = jnp.dot(hk.astype(_BF16), ur[...],
                     preferred_element_type=_F32) + bur[...]
        sk = hk / (1.0 + jnp.exp(nr1 - uk))
        sum_h = hk if sum_h is None else sum_h + hk
        sgh = sk if sgh is None else sgh + sk
    z = _sigmoid(_unpack_cols(pzh[:, 0:HALF])
                 + jnp.dot(sum_h.astype(_BF16), wz2[...],
                           preferred_element_type=_F32))
    ph = jnp.tanh(_unpack_cols(pzh[:, 2 * HALF:])
                  + jnp.dot(sgh.astype(_BF16), wh2[...],
                            preferred_element_type=_F32))
    h = (1.0 - z) * sum_h + z * ph
    h = jnp.where(_row_mask(block, blk_off), h, 0.0)
    out[...] = _pack_cols(h)


def _readout_kernel(g0, g1, g2, g3, pre_o, wo2, out, *, block):
    nei = None
    for g in (g0, g1, g2, g3):
        hk = _unpack_cols(g[...])
        nei = hk if nei is None else nei + hk
    y = pre_o[...] + jnp.dot(nei.astype(_BF16), wo2[...],
                             preferred_element_type=_F32)
    y = jnp.maximum(y, 0.0)
    out[...] = jnp.where(_row_mask(block), y, 0.0)


def _full(spec_shape):
    return pl.BlockSpec(spec_shape, lambda i: tuple(0 for _ in spec_shape))


def kernel(fnode, fmess, agraph, bgraph, scope, W_z, b_z, W_r, U_r, b_ur,
           W_h, b_h, W_o, b_o):
    del scope
    perm = jnp.concatenate([jnp.arange(0, HIDDEN, 2, dtype=jnp.int32),
                            jnp.arange(1, HIDDEN, 2, dtype=jnp.int32)])
    Wz1, Wz2 = W_z[:INPUT_SIZE][:, perm], W_z[INPUT_SIZE:][perm][:, perm]
    Wh1, Wh2 = W_h[:INPUT_SIZE][:, perm], W_h[INPUT_SIZE:][perm][:, perm]
    Wo1, Wo2 = W_o[:ATOM_FDIM], W_o[ATOM_FDIM:][perm]
    Urp = U_r[perm][:, perm]
    Wrp = W_r[:, perm]
    bz = b_z[perm].reshape(1, HIDDEN)
    bh = b_h[perm].reshape(1, HIDDEN)
    bo = b_o.reshape(1, HIDDEN)
    bur = b_ur[perm].reshape(1, HIDDEN)

    fmess_pad = jnp.pad(fmess, ((0, E2 - N_EDGES), (0, 0)))
    src_idx = fmess_pad[:, 0].reshape(-1, _WIN)
    EH = E2 // 2
    bgraph_pad = jnp.pad(bgraph, ((0, E2 - N_EDGES), (0, 0)))
    bgA_idx = bgraph_pad[:EH].T.reshape(-1, _WIN)
    bgB_idx = bgraph_pad[EH:].T.reshape(-1, _WIN)
    ag_flat = agraph.T.reshape(-1)
    ag_pad = 16 * _NBUF * _WIN * 2
    ag_total = ((ag_flat.shape[0] + ag_pad - 1) // ag_pad) * ag_pad
    ag_idx = jnp.pad(ag_flat, (0, ag_total - ag_flat.shape[0]))
    ag_idx = ag_idx.reshape(-1, _WIN)
    fnode_pad = jnp.pad(fnode, ((0, 0), (0, 126)))

    fsrc = _sc_gather_rows(fnode_pad, src_idx, skew=1.0)

    pzh = pl.pallas_call(
        _edge_pre_kernel,
        grid=(E2 // BU,),
        in_specs=[
            pl.BlockSpec((BU, 128), lambda i: (i, 0)),
            pl.BlockSpec((BU, 3), lambda i: (i, 0)),
            _full((INPUT_SIZE, HIDDEN)),
            _full((INPUT_SIZE, HIDDEN)),
            _full((INPUT_SIZE, HIDDEN)),
            _full((1, HIDDEN)),
            _full((1, HIDDEN)),
        ],
        out_specs=pl.BlockSpec((BU, 3 * HALF), lambda i: (i, 0)),
        out_shape=jax.ShapeDtypeStruct((E2, 3 * HALF), jnp.int32),
    )(fsrc, fmess_pad, Wz1, Wrp, Wh1, bz, bh)

    pre_o = pl.pallas_call(
        _node_pre_kernel,
        grid=(N_NODES // BR,),
        in_specs=[
            pl.BlockSpec((BR, 2), lambda i: (i, 0)),
            _full((ATOM_FDIM, HIDDEN)),
            _full((1, HIDDEN)),
        ],
        out_specs=pl.BlockSpec((BR, HIDDEN), lambda i: (i, 0)),
        out_shape=jax.ShapeDtypeStruct((N_NODES, HIDDEN), _F32),
    )(fnode, Wo1, bo)

    c = pl.pallas_call(
        functools.partial(_first_iter_kernel, block=BU),
        grid=(E2 // BU,),
        in_specs=[pl.BlockSpec((BU, 3 * HALF), lambda i: (i, 0))],
        out_specs=pl.BlockSpec((BU, HALF), lambda i: (i, 0)),
        out_shape=jax.ShapeDtypeStruct((E2, HALF), jnp.int32),
    )(pzh)

    Wz2b = Wz2.astype(_BF16)
    Wh2b = Wh2.astype(_BF16)
    Urpb = Urp.astype(_BF16)
    nsegh = EH // BU

    def upd_half(g, half, prev):
        blk_off = half * nsegh
        gspecs = [
            pl.BlockSpec((BU, HALF),
                         functools.partial(lambda k, i: (i + k * nsegh, 0), k))
            for k in range(MAX_NB)
        ]
        dspec = pl.BlockSpec((BU, HALF), lambda i: (i + blk_off, 0))
        other = [
            pl.BlockSpec((BU, 3 * HALF), lambda i: (i + blk_off, 0)),
            _full((HIDDEN, HIDDEN)),
            _full((HIDDEN, HIDDEN)),
            _full((HIDDEN, HIDDEN)),
            _full((1, HIDDEN)),
        ]
        args = (g, g, g, g, pzh, Wz2b, Wh2b, Urpb, bur)
        aliases = {}
        if prev is not None:
            other = other + [dspec]
            args = args + (prev,)
            aliases = {9: 0}
        return pl.pallas_call(
            functools.partial(_update_kernel, block=BU, blk_off=blk_off),
            grid=(EH // BU,),
            in_specs=gspecs + other,
            out_specs=dspec,
            out_shape=jax.ShapeDtypeStruct((E2, HALF), jnp.int32),
            input_output_aliases=aliases,
        )(*args)

    for _ in range(DEPTH - 1):
        ga = _sc_gather_rows(c, bgA_idx, skew=1.0)
        gb = _sc_gather_rows(c, bgB_idx, skew=1.0)
        ca = upd_half(ga, 0, None)
        c = upd_half(gb, 1, ca)

    gn = _sc_gather_rows(c, ag_idx, skew=1.0)
    nsegn = N_NODES // BR
    out = pl.pallas_call(
        functools.partial(_readout_kernel, block=BR),
        grid=(N_NODES // BR,),
        in_specs=[
            pl.BlockSpec((BR, HALF),
                         functools.partial(lambda k, i: (i + k * nsegn, 0), k))
            for k in range(MAX_NB)
        ] + [
            pl.BlockSpec((BR, HIDDEN), lambda i: (i, 0)),
            _full((HIDDEN, HIDDEN)),
        ],
        out_specs=pl.BlockSpec((BR, HIDDEN), lambda i: (i, 0)),
        out_shape=jax.ShapeDtypeStruct((N_NODES, HIDDEN), _F32),
    )(gn, gn, gn, gn, pre_o, Wo2.astype(_BF16))
    return out
